# baseline (speedup 1.0000x reference)
import functools

import jax
import jax.numpy as jnp
from jax import lax
from jax.experimental import pallas as pl
from jax.experimental.pallas import tpu as pltpu
from jax.experimental.pallas import tpu_sc as plsc

N = 10000
E = 320000
D1 = 128
D2 = 40

PN = 10240
NW = 32
EPW = E // NW
CH = 125
NCH = EPW // CH
NHLF = NCH // 2
RPT = PN // 16


@functools.cache
def _get_deg_kernel():
    mesh = plsc.VectorSubcoreMesh(core_axis_name="c", subcore_axis_name="s")

    @functools.partial(
        pl.kernel,
        out_type=jax.ShapeDtypeStruct((2, PN, D1), jnp.float32),
        mesh=mesh,
        scratch_types=[
            pltpu.VMEM((NCH, CH), jnp.int32),
            pltpu.VMEM((CH, D1), jnp.float32),
            pltpu.VMEM_SHARED((PN, D1), jnp.float32),
        ],
    )
    def deg_kernel(col_hbm, ones_hbm, z_hbm, out_hbm, colv, onesv, acc):
        cid = lax.axis_index("c")
        sid = lax.axis_index("s")
        wid = sid * 2 + cid
        pltpu.sync_copy(z_hbm.at[pl.ds(sid * RPT, RPT)],
                        acc.at[pl.ds(sid * RPT, RPT)])
        pltpu.sync_copy(ones_hbm, onesv)
        pltpu.sync_copy(col_hbm.at[wid], colv)
        plsc.subcore_barrier()

        def body(j, carry):
            pltpu.sync_copy(onesv, acc.at[colv.at[j]], add=True)
            return carry

        lax.fori_loop(0, NCH, body, 0)
        plsc.subcore_barrier()
        pltpu.sync_copy(acc.at[pl.ds(sid * RPT, RPT)],
                        out_hbm.at[cid, pl.ds(sid * RPT, RPT)])

    return deg_kernel


@functools.cache
def _get_edge_kernel():
    mesh = plsc.VectorSubcoreMesh(core_axis_name="c", subcore_axis_name="s")

    @functools.partial(
        pl.kernel,
        out_type=jax.ShapeDtypeStruct((2, PN, D1), jnp.float32),
        mesh=mesh,
        scratch_types=[
            pltpu.VMEM((NHLF, CH), jnp.int32),
            pltpu.VMEM((NHLF, CH), jnp.int32),
            pltpu.VMEM((CH, D1), jnp.float32),
            pltpu.VMEM((CH, D1), jnp.float32),
            pltpu.VMEM_SHARED((PN, D1), jnp.float32),
            pltpu.SemaphoreType.DMA,
            pltpu.SemaphoreType.DMA,
        ],
    )
    def edge_kernel(g_hbm, row_hbm, col_hbm, z_hbm, out_hbm,
                    rowv, colv, buf0, buf1, acc, sem0, sem1):
        cid = lax.axis_index("c")
        sid = lax.axis_index("s")
        wid = sid * 2 + cid
        pltpu.sync_copy(z_hbm.at[pl.ds(sid * RPT, RPT)],
                        acc.at[pl.ds(sid * RPT, RPT)])
        plsc.subcore_barrier()

        for h in range(2):
            pltpu.sync_copy(row_hbm.at[wid, pl.ds(h * NHLF, NHLF)], rowv)
            pltpu.sync_copy(col_hbm.at[wid, pl.ds(h * NHLF, NHLF)], colv)

            pltpu.async_copy(g_hbm.at[rowv.at[0]], buf0, sem0)

            def body(t, carry):
                j = 2 * t
                pltpu.async_copy(g_hbm.at[rowv.at[j + 1]], buf1, sem1)
                pltpu.make_async_copy(g_hbm.at[rowv.at[j]], buf0, sem0).wait()
                pltpu.sync_copy(buf0, acc.at[colv.at[j]], add=True)

                @pl.when(j + 2 < NHLF)
                def _():
                    pltpu.async_copy(g_hbm.at[rowv.at[j + 2]], buf0, sem0)

                pltpu.make_async_copy(g_hbm.at[rowv.at[j + 1]], buf1,
                                      sem1).wait()
                pltpu.sync_copy(buf1, acc.at[colv.at[j + 1]], add=True)
                return carry

            lax.fori_loop(0, NHLF // 2, body, 0)
        plsc.subcore_barrier()
        pltpu.sync_copy(acc.at[pl.ds(sid * RPT, RPT)],
                        out_hbm.at[cid, pl.ds(sid * RPT, RPT)])

    return edge_kernel


_BR = 1000


def _prep_body(x_ref, w_ref, degp_ref, g1_ref, dinv_ref):
    deg = degp_ref[0, :, 0:8] + degp_ref[1, :, 0:8] + 1.0
    dinv = lax.rsqrt(deg)
    dinv_ref[...] = dinv
    g1_ref[...] = jnp.dot(x_ref[...], w_ref[...],
                          preferred_element_type=jnp.float32) * dinv[:, 0:1]


def _mid_body(s1p_ref, g1_ref, dinv_ref, b1_ref, w2_ref, feat_ref, g2_ref):
    dinv = dinv_ref[...][:, 0:1]
    pre = (s1p_ref[0] + s1p_ref[1] + g1_ref[...]) * dinv + b1_ref[...]
    feat = jnp.maximum(pre, 0.0)
    feat_ref[...] = feat
    g2_ref[...] = jnp.dot(feat, w2_ref[...],
                          preferred_element_type=jnp.float32) * dinv


def _out_body(s2p_ref, g2_ref, dinv_ref, b2_ref, o_ref):
    s = (s2p_ref[0] + s2p_ref[1] + g2_ref[...])[:, :D2]
    z = s * dinv_ref[...][:, 0:1] + b2_ref[...]
    m = jnp.max(z, axis=1, keepdims=True)
    lse = jnp.log(jnp.sum(jnp.exp(z - m), axis=1, keepdims=True)) + m
    o_ref[...] = z - lse


def kernel(x, edge_index, W1, b1, W2, b2):
    row2d = edge_index[0].reshape(NW, NCH, CH)
    col2d = edge_index[1].reshape(NW, NCH, CH)
    z128 = jnp.zeros((PN, D1), jnp.float32)
    W2p = jnp.pad(W2, ((0, 0), (0, D1 - D2)))
    xp = jnp.pad(x, ((0, PN - N), (0, 0)))

    ones128 = jnp.ones((CH, D1), jnp.float32)
    degp = _get_deg_kernel()(col2d, ones128, z128)

    g1, dinv = pl.pallas_call(
        _prep_body,
        grid=(PN // 1024,),
        in_specs=[
            pl.BlockSpec((1024, D1), lambda i: (i, 0)),
            pl.BlockSpec((D1, D1), lambda i: (0, 0)),
            pl.BlockSpec((2, 1024, D1), lambda i: (0, i, 0)),
        ],
        out_specs=[
            pl.BlockSpec((1024, D1), lambda i: (i, 0)),
            pl.BlockSpec((1024, 8), lambda i: (i, 0)),
        ],
        out_shape=[
            jax.ShapeDtypeStruct((PN, D1), jnp.float32),
            jax.ShapeDtypeStruct((PN, 8), jnp.float32),
        ],
    )(xp, W1, degp)

    s1p = _get_edge_kernel()(g1, row2d, col2d, z128)

    feat, g2 = pl.pallas_call(
        _mid_body,
        grid=(N // _BR,),
        in_specs=[
            pl.BlockSpec((2, _BR, D1), lambda i: (0, i, 0)),
            pl.BlockSpec((_BR, D1), lambda i: (i, 0)),
            pl.BlockSpec((_BR, 8), lambda i: (i, 0)),
            pl.BlockSpec((1, D1), lambda i: (0, 0)),
            pl.BlockSpec((D1, D1), lambda i: (0, 0)),
        ],
        out_specs=[
            pl.BlockSpec((_BR, D1), lambda i: (i, 0)),
            pl.BlockSpec((_BR, D1), lambda i: (i, 0)),
        ],
        out_shape=[
            jax.ShapeDtypeStruct((N, D1), jnp.float32),
            jax.ShapeDtypeStruct((PN, D1), jnp.float32),
        ],
    )(s1p, g1, dinv, b1.reshape(1, D1), W2p)

    s2p = _get_edge_kernel()(g2, row2d, col2d, z128)

    logp = pl.pallas_call(
        _out_body,
        grid=(N // _BR,),
        in_specs=[
            pl.BlockSpec((2, _BR, D1), lambda i: (0, i, 0)),
            pl.BlockSpec((_BR, D1), lambda i: (i, 0)),
            pl.BlockSpec((_BR, 8), lambda i: (i, 0)),
            pl.BlockSpec((1, D2), lambda i: (0, 0)),
        ],
        out_specs=pl.BlockSpec((_BR, D2), lambda i: (i, 0)),
        out_shape=jax.ShapeDtypeStruct((N, D2), jnp.float32),
    )(s2p, g2, dinv, b2.reshape(1, D2))

    return logp, feat


# speedup vs baseline: 1.0098x; 1.0098x over previous
import functools

import jax
import jax.numpy as jnp
from jax import lax
from jax.experimental import pallas as pl
from jax.experimental.pallas import tpu as pltpu
from jax.experimental.pallas import tpu_sc as plsc

N = 10000
E = 320000
D1 = 128
D2 = 40

PN = 10240
NW = 32
EPW = E // NW
CH = 125
NCH = EPW // CH
NHLF = NCH // 2
RPT = PN // 16


@functools.cache
def _get_deg_kernel():
    mesh = plsc.VectorSubcoreMesh(core_axis_name="c", subcore_axis_name="s")

    @functools.partial(
        pl.kernel,
        out_type=jax.ShapeDtypeStruct((2, PN, D1), jnp.float32),
        mesh=mesh,
        scratch_types=[
            pltpu.VMEM((NCH, CH), jnp.int32),
            pltpu.VMEM((CH, D1), jnp.float32),
            pltpu.VMEM_SHARED((PN, D1), jnp.float32),
        ],
    )
    def deg_kernel(col_hbm, ones_hbm, z_hbm, out_hbm, colv, onesv, acc):
        cid = lax.axis_index("c")
        sid = lax.axis_index("s")
        wid = sid * 2 + cid
        pltpu.sync_copy(z_hbm.at[pl.ds(sid * RPT, RPT)],
                        acc.at[pl.ds(sid * RPT, RPT)])
        pltpu.sync_copy(ones_hbm, onesv)
        pltpu.sync_copy(col_hbm.at[wid], colv)
        plsc.subcore_barrier()

        def body(j, carry):
            pltpu.sync_copy(onesv, acc.at[colv.at[j]], add=True)
            return carry

        lax.fori_loop(0, NCH, body, 0)
        plsc.subcore_barrier()
        pltpu.sync_copy(acc.at[pl.ds(sid * RPT, RPT)],
                        out_hbm.at[cid, pl.ds(sid * RPT, RPT)])

    return deg_kernel


@functools.cache
def _get_edge_kernel():
    mesh = plsc.VectorSubcoreMesh(core_axis_name="c", subcore_axis_name="s")

    @functools.partial(
        pl.kernel,
        out_type=jax.ShapeDtypeStruct((2, PN, D1), jnp.float32),
        mesh=mesh,
        scratch_types=[
            pltpu.VMEM((NHLF, CH), jnp.int32),
            pltpu.VMEM((NHLF, CH), jnp.int32),
            pltpu.VMEM((CH, D1), jnp.float32),
            pltpu.VMEM((CH, D1), jnp.float32),
            pltpu.VMEM_SHARED((PN, D1), jnp.float32),
            pltpu.SemaphoreType.DMA,
            pltpu.SemaphoreType.DMA,
        ],
    )
    def edge_kernel(g_hbm, row_hbm, col_hbm, z_hbm, out_hbm,
                    rowv, colv, buf0, buf1, acc, sem0, sem1):
        cid = lax.axis_index("c")
        sid = lax.axis_index("s")
        wid = sid * 2 + cid
        pltpu.async_copy(z_hbm.at[pl.ds(sid * RPT, RPT)],
                         acc.at[pl.ds(sid * RPT, RPT)], sem0)
        pltpu.sync_copy(row_hbm.at[wid, pl.ds(0, NHLF)], rowv)
        pltpu.sync_copy(col_hbm.at[wid, pl.ds(0, NHLF)], colv)
        pltpu.make_async_copy(z_hbm.at[pl.ds(sid * RPT, RPT)],
                              acc.at[pl.ds(sid * RPT, RPT)], sem0).wait()
        plsc.subcore_barrier()

        for h in range(2):
            if h:
                pltpu.sync_copy(row_hbm.at[wid, pl.ds(h * NHLF, NHLF)], rowv)
                pltpu.sync_copy(col_hbm.at[wid, pl.ds(h * NHLF, NHLF)], colv)

            pltpu.async_copy(g_hbm.at[rowv.at[0]], buf0, sem0)
            pltpu.async_copy(g_hbm.at[rowv.at[1]], buf1, sem1)

            def body(t, carry):
                j = 4 * t
                for u in range(4):
                    bu = buf0 if u % 2 == 0 else buf1
                    su = sem0 if u % 2 == 0 else sem1
                    pltpu.make_async_copy(g_hbm.at[rowv.at[j + u]], bu,
                                          su).wait()
                    pltpu.sync_copy(bu, acc.at[colv.at[j + u]], add=True)

                    @pl.when(j + u + 2 < NHLF)
                    def _():
                        pltpu.async_copy(g_hbm.at[rowv.at[j + u + 2]], bu, su)

                return carry

            lax.fori_loop(0, NHLF // 4, body, 0)
        plsc.subcore_barrier()
        pltpu.sync_copy(acc.at[pl.ds(sid * RPT, RPT)],
                        out_hbm.at[cid, pl.ds(sid * RPT, RPT)])

    return edge_kernel


_BR = 1000


def _prep_body(x_ref, w_ref, degp_ref, g1_ref, dinv_ref):
    deg = degp_ref[0, :, 0:8] + degp_ref[1, :, 0:8] + 1.0
    dinv = lax.rsqrt(deg)
    dinv_ref[...] = dinv
    g1_ref[...] = jnp.dot(x_ref[...], w_ref[...],
                          preferred_element_type=jnp.float32) * dinv[:, 0:1]


def _mid_body(s1p_ref, g1_ref, dinv_ref, b1_ref, w2_ref, feat_ref, g2_ref):
    dinv = dinv_ref[...][:, 0:1]
    pre = (s1p_ref[0] + s1p_ref[1] + g1_ref[...]) * dinv + b1_ref[...]
    feat = jnp.maximum(pre, 0.0)
    feat_ref[...] = feat
    g2_ref[...] = jnp.dot(feat, w2_ref[...],
                          preferred_element_type=jnp.float32) * dinv


def _out_body(s2p_ref, g2_ref, dinv_ref, b2_ref, o_ref):
    s = (s2p_ref[0] + s2p_ref[1] + g2_ref[...])[:, :D2]
    z = s * dinv_ref[...][:, 0:1] + b2_ref[...]
    m = jnp.max(z, axis=1, keepdims=True)
    lse = jnp.log(jnp.sum(jnp.exp(z - m), axis=1, keepdims=True)) + m
    o_ref[...] = z - lse


def kernel(x, edge_index, W1, b1, W2, b2):
    row2d = edge_index[0].reshape(NW, NCH, CH)
    col2d = edge_index[1].reshape(NW, NCH, CH)
    z128 = jnp.zeros((PN, D1), jnp.float32)
    W2p = jnp.pad(W2, ((0, 0), (0, D1 - D2)))
    xp = jnp.pad(x, ((0, PN - N), (0, 0)))

    ones128 = jnp.ones((CH, D1), jnp.float32)
    degp = _get_deg_kernel()(col2d, ones128, z128)

    g1, dinv = pl.pallas_call(
        _prep_body,
        grid=(PN // 1024,),
        in_specs=[
            pl.BlockSpec((1024, D1), lambda i: (i, 0)),
            pl.BlockSpec((D1, D1), lambda i: (0, 0)),
            pl.BlockSpec((2, 1024, D1), lambda i: (0, i, 0)),
        ],
        out_specs=[
            pl.BlockSpec((1024, D1), lambda i: (i, 0)),
            pl.BlockSpec((1024, 8), lambda i: (i, 0)),
        ],
        out_shape=[
            jax.ShapeDtypeStruct((PN, D1), jnp.float32),
            jax.ShapeDtypeStruct((PN, 8), jnp.float32),
        ],
    )(xp, W1, degp)

    s1p = _get_edge_kernel()(g1, row2d, col2d, z128)

    feat, g2 = pl.pallas_call(
        _mid_body,
        grid=(N // _BR,),
        in_specs=[
            pl.BlockSpec((2, _BR, D1), lambda i: (0, i, 0)),
            pl.BlockSpec((_BR, D1), lambda i: (i, 0)),
            pl.BlockSpec((_BR, 8), lambda i: (i, 0)),
            pl.BlockSpec((1, D1), lambda i: (0, 0)),
            pl.BlockSpec((D1, D1), lambda i: (0, 0)),
        ],
        out_specs=[
            pl.BlockSpec((_BR, D1), lambda i: (i, 0)),
            pl.BlockSpec((_BR, D1), lambda i: (i, 0)),
        ],
        out_shape=[
            jax.ShapeDtypeStruct((N, D1), jnp.float32),
            jax.ShapeDtypeStruct((PN, D1), jnp.float32),
        ],
    )(s1p, g1, dinv, b1.reshape(1, D1), W2p)

    s2p = _get_edge_kernel()(g2, row2d, col2d, z128)

    logp = pl.pallas_call(
        _out_body,
        grid=(N // _BR,),
        in_specs=[
            pl.BlockSpec((2, _BR, D1), lambda i: (0, i, 0)),
            pl.BlockSpec((_BR, D1), lambda i: (i, 0)),
            pl.BlockSpec((_BR, 8), lambda i: (i, 0)),
            pl.BlockSpec((1, D2), lambda i: (0, 0)),
        ],
        out_specs=pl.BlockSpec((_BR, D2), lambda i: (i, 0)),
        out_shape=jax.ShapeDtypeStruct((N, D2), jnp.float32),
    )(s2p, g2, dinv, b2.reshape(1, D2))

    return logp, feat
